# baseline (speedup 1.0000x reference)
import functools

import jax
import jax.numpy as jnp
from jax import lax
from jax.experimental import pallas as pl
from jax.experimental.pallas import tpu as pltpu
from jax.experimental.pallas import tpu_sc as plsc

NSPLIT = 4


def _sc_gather(table, idx_flat, Q, c0, qn, chunk=400):
    V, D = table.shape
    info = plsc.get_sparse_core_info()
    nw = info.num_cores * info.num_subcores
    g_per_w = qn // nw
    n_chunks = g_per_w // chunk
    mesh = plsc.VectorSubcoreMesh(core_axis_name="c", subcore_axis_name="s")

    @functools.partial(
        pl.kernel,
        mesh=mesh,
        out_type=jax.ShapeDtypeStruct((qn, 4 * D), jnp.float32),
        scratch_types=[
            pltpu.VMEM((4 * chunk,), jnp.int32),
            pltpu.VMEM((4 * chunk, D), jnp.float32),
            pltpu.SemaphoreType.DMA,
        ],
        compiler_params=pltpu.CompilerParams(use_tc_tiling_on_sc=False),
    )
    def gather_kernel(table_hbm, idx_hbm, out_hbm, idx_v, rows_v, sem):
        wid = lax.axis_index("s") * info.num_cores + lax.axis_index("c")
        base = wid * g_per_w

        def body(i, carry):
            g0 = base + i * chunk
            for r in range(4):
                pltpu.sync_copy(
                    idx_hbm.at[pl.ds(r * Q + c0 + g0, chunk)],
                    idx_v.at[pl.ds(r * chunk, chunk)],
                )
            copies = [
                pltpu.async_copy(
                    table_hbm.at[idx_v.at[pl.ds(r * chunk, chunk)]],
                    rows_v.at[pl.ds(r * chunk, chunk)],
                    sem,
                )
                for r in range(4)
            ]
            for c in copies:
                c.wait()
            for r in range(4):
                pltpu.sync_copy(
                    rows_v.at[pl.ds(r * chunk, chunk)],
                    out_hbm.at[pl.ds(g0, chunk), pl.ds(r * D, D)],
                )
            return carry

        lax.fori_loop(0, n_chunks, body, 0)

    return gather_kernel(table, idx_flat)


def _tc_project(emb2, prev, w_t, bias, Q, c0, block_m=2048):
    qn = emb2.shape[0]
    D, O = w_t.shape
    o0 = c0 // block_m

    def body(emb_ref, prev_ref, wt_ref, b_ref, out_ref):
        del prev_ref
        for r in range(4):
            out_ref[r] = (
                jnp.dot(
                    emb_ref[:, r * D : (r + 1) * D],
                    wt_ref[...],
                    preferred_element_type=jnp.float32,
                )
                + b_ref[...]
            )

    args = [emb2] + ([prev] if prev is not None else []) + [w_t, bias.reshape(1, O)]
    prev_specs = [pl.BlockSpec(memory_space=pl.ANY)] if prev is not None else []
    if prev is None:
        def body0(emb_ref, wt_ref, b_ref, out_ref):
            body(emb_ref, None, wt_ref, b_ref, out_ref)
        kernel_body = body0
        aliases = {}
    else:
        kernel_body = body
        aliases = {1: 0}

    return pl.pallas_call(
        kernel_body,
        grid=(qn // block_m,),
        in_specs=[pl.BlockSpec((block_m, 4 * D), lambda i: (i, 0))]
        + prev_specs
        + [
            pl.BlockSpec((D, O), lambda i: (0, 0)),
            pl.BlockSpec((1, O), lambda i: (0, 0)),
        ],
        out_specs=pl.BlockSpec((4, block_m, O), lambda i: (0, o0 + i, 0)),
        out_shape=jax.ShapeDtypeStruct((4, Q, O), jnp.float32),
        input_output_aliases=aliases,
    )(*args)


def kernel(idx, table, W, b):
    Bb, L = idx.shape
    O = W.shape[0]
    B = Bb * L
    Q = B // 4
    qn = Q // NSPLIT
    idx_perm = idx.T.reshape(-1).astype(jnp.int32)
    w_t = W.T
    out = None
    for c in range(NSPLIT):
        emb2_c = _sc_gather(table, idx_perm, Q, c * qn, qn)
        out = _tc_project(emb2_c, out, w_t, b, Q, c * qn)
    return out.reshape(L, Bb, O).transpose(1, 0, 2)


# speedup vs baseline: 1.0255x; 1.0255x over previous
import functools

import jax
import jax.numpy as jnp
from jax import lax
from jax.experimental import pallas as pl
from jax.experimental.pallas import tpu as pltpu
from jax.experimental.pallas import tpu_sc as plsc

NSPLIT = 2


def _sc_gather(table, idx_flat, Q, c0, qn, chunk=800):
    V, D = table.shape
    info = plsc.get_sparse_core_info()
    nw = info.num_cores * info.num_subcores
    g_per_w = qn // nw
    n_chunks = g_per_w // chunk
    mesh = plsc.VectorSubcoreMesh(core_axis_name="c", subcore_axis_name="s")

    @functools.partial(
        pl.kernel,
        mesh=mesh,
        out_type=jax.ShapeDtypeStruct((qn, 4 * D), jnp.float32),
        scratch_types=[
            pltpu.VMEM((4 * chunk,), jnp.int32),
            pltpu.VMEM((4 * chunk, D), jnp.float32),
            pltpu.SemaphoreType.DMA,
        ],
        compiler_params=pltpu.CompilerParams(use_tc_tiling_on_sc=False),
    )
    def gather_kernel(table_hbm, idx_hbm, out_hbm, idx_v, rows_v, sem):
        wid = lax.axis_index("s") * info.num_cores + lax.axis_index("c")
        base = wid * g_per_w

        def body(i, carry):
            g0 = base + i * chunk
            for r in range(4):
                pltpu.sync_copy(
                    idx_hbm.at[pl.ds(r * Q + c0 + g0, chunk)],
                    idx_v.at[pl.ds(r * chunk, chunk)],
                )
            copies = [
                pltpu.async_copy(
                    table_hbm.at[idx_v.at[pl.ds(r * chunk, chunk)]],
                    rows_v.at[pl.ds(r * chunk, chunk)],
                    sem,
                )
                for r in range(4)
            ]
            for c in copies:
                c.wait()
            for r in range(4):
                pltpu.sync_copy(
                    rows_v.at[pl.ds(r * chunk, chunk)],
                    out_hbm.at[pl.ds(g0, chunk), pl.ds(r * D, D)],
                )
            return carry

        lax.fori_loop(0, n_chunks, body, 0)

    return gather_kernel(table, idx_flat)


def _tc_project(emb2, prev, w_t, bias, Q, c0, block_m=4096):
    qn = emb2.shape[0]
    D, O = w_t.shape
    o0 = c0 // block_m

    def body(emb_ref, prev_ref, wt_ref, b_ref, out_ref):
        del prev_ref
        for r in range(4):
            out_ref[r] = (
                jnp.dot(
                    emb_ref[:, r * D : (r + 1) * D],
                    wt_ref[...],
                    preferred_element_type=jnp.float32,
                )
                + b_ref[...]
            )

    args = [emb2] + ([prev] if prev is not None else []) + [w_t, bias.reshape(1, O)]
    prev_specs = [pl.BlockSpec(memory_space=pl.ANY)] if prev is not None else []
    if prev is None:
        def body0(emb_ref, wt_ref, b_ref, out_ref):
            body(emb_ref, None, wt_ref, b_ref, out_ref)
        kernel_body = body0
        aliases = {}
    else:
        kernel_body = body
        aliases = {1: 0}

    return pl.pallas_call(
        kernel_body,
        grid=(qn // block_m,),
        in_specs=[pl.BlockSpec((block_m, 4 * D), lambda i: (i, 0))]
        + prev_specs
        + [
            pl.BlockSpec((D, O), lambda i: (0, 0)),
            pl.BlockSpec((1, O), lambda i: (0, 0)),
        ],
        out_specs=pl.BlockSpec((4, block_m, O), lambda i: (0, o0 + i, 0)),
        out_shape=jax.ShapeDtypeStruct((4, Q, O), jnp.float32),
        input_output_aliases=aliases,
    )(*args)


def kernel(idx, table, W, b):
    Bb, L = idx.shape
    O = W.shape[0]
    B = Bb * L
    Q = B // 4
    qn = Q // NSPLIT
    idx_perm = idx.T.reshape(-1).astype(jnp.int32)
    w_t = W.T
    out = None
    for c in range(NSPLIT):
        emb2_c = _sc_gather(table, idx_perm, Q, c * qn, qn)
        out = _tc_project(emb2_c, out, w_t, b, Q, c * qn)
    return out.reshape(L, Bb, O).transpose(1, 0, 2)


# speedup vs baseline: 1.0285x; 1.0029x over previous
import functools

import jax
import jax.numpy as jnp
from jax import lax
from jax.experimental import pallas as pl
from jax.experimental.pallas import tpu as pltpu
from jax.experimental.pallas import tpu_sc as plsc

NSPLIT = 2


def _sc_gather(table, idx_flat, Q, c0, qn, chunk=800):
    V, D = table.shape
    info = plsc.get_sparse_core_info()
    nw = info.num_cores * info.num_subcores
    g_per_w = qn // nw
    n_chunks = g_per_w // chunk
    mesh = plsc.VectorSubcoreMesh(core_axis_name="c", subcore_axis_name="s")

    @functools.partial(
        pl.kernel,
        mesh=mesh,
        out_type=jax.ShapeDtypeStruct((qn, 4 * D), jnp.float32),
        scratch_types=[
            pltpu.VMEM((4 * chunk,), jnp.int32),
            pltpu.VMEM((4 * chunk, D), jnp.float32),
            pltpu.SemaphoreType.DMA,
        ],
        compiler_params=pltpu.CompilerParams(use_tc_tiling_on_sc=False),
    )
    def gather_kernel(table_hbm, idx_hbm, out_hbm, idx_v, rows_v, sem):
        wid = lax.axis_index("s") * info.num_cores + lax.axis_index("c")
        base = wid * g_per_w

        def body(i, carry):
            g0 = base + i * chunk
            for r in range(4):
                pltpu.sync_copy(
                    idx_hbm.at[pl.ds(r * Q + c0 + g0, chunk)],
                    idx_v.at[pl.ds(r * chunk, chunk)],
                )
            copies = [
                pltpu.async_copy(
                    table_hbm.at[idx_v.at[pl.ds(r * chunk, chunk)]],
                    rows_v.at[pl.ds(r * chunk, chunk)],
                    sem,
                )
                for r in range(4)
            ]
            for c in copies:
                c.wait()
            for r in range(4):
                pltpu.sync_copy(
                    rows_v.at[pl.ds(r * chunk, chunk)],
                    out_hbm.at[pl.ds(g0, chunk), pl.ds(r * D, D)],
                )
            return carry

        lax.fori_loop(0, n_chunks, body, 0)

    return gather_kernel(table, idx_flat)


def _tc_project(emb2, prev, w_t, bias, Q, c0, block_m=4096, nbuf=4):
    qn = emb2.shape[0]
    D, O = w_t.shape
    nblk = qn // block_m

    def body(emb_ref, prev_ref, wt_ref, b_ref, out_ref, acc, sems):
        del prev_ref
        i = pl.program_id(0)
        slot = lax.rem(i, nbuf)
        row0 = c0 + i * block_m

        def _mk(s):
            return pltpu.make_async_copy(
                acc.at[s],
                out_ref.at[:, pl.ds(row0 + (s - slot) * block_m, block_m), :],
                sems.at[s],
            )

        @pl.when(i >= nbuf)
        def _wait_old():
            pltpu.make_async_copy(
                acc.at[slot],
                out_ref.at[:, pl.ds(row0 - nbuf * block_m, block_m), :],
                sems.at[slot],
            ).wait()

        for r in range(4):
            acc[slot, r] = (
                jnp.dot(
                    emb_ref[:, r * D : (r + 1) * D],
                    wt_ref[...],
                    preferred_element_type=jnp.float32,
                )
                + b_ref[...]
            )
        _mk(slot).start()

        @pl.when(i == nblk - 1)
        def _drain():
            for k in range(nbuf):
                s = lax.rem(i - k, nbuf)
                pltpu.make_async_copy(
                    acc.at[s],
                    out_ref.at[:, pl.ds(row0 - k * block_m, block_m), :],
                    sems.at[s],
                ).wait()

    args = [emb2] + ([prev] if prev is not None else []) + [w_t, bias.reshape(1, O)]
    prev_specs = [pl.BlockSpec(memory_space=pl.ANY)] if prev is not None else []
    if prev is None:
        def body0(emb_ref, wt_ref, b_ref, out_ref, acc, sems):
            body(emb_ref, None, wt_ref, b_ref, out_ref, acc, sems)
        kernel_body = body0
        aliases = {}
    else:
        kernel_body = body
        aliases = {1: 0}

    return pl.pallas_call(
        kernel_body,
        grid=(nblk,),
        in_specs=[pl.BlockSpec((block_m, 4 * D), lambda i: (i, 0))]
        + prev_specs
        + [
            pl.BlockSpec((D, O), lambda i: (0, 0)),
            pl.BlockSpec((1, O), lambda i: (0, 0)),
        ],
        out_specs=pl.BlockSpec(memory_space=pl.ANY),
        out_shape=jax.ShapeDtypeStruct((4, Q, O), jnp.float32),
        scratch_shapes=[
            pltpu.VMEM((nbuf, 4, block_m, O), jnp.float32),
            pltpu.SemaphoreType.DMA((nbuf,)),
        ],
        input_output_aliases=aliases,
    )(*args)


def kernel(idx, table, W, b):
    Bb, L = idx.shape
    O = W.shape[0]
    B = Bb * L
    Q = B // 4
    qn = Q // NSPLIT
    idx_perm = idx.T.reshape(-1).astype(jnp.int32)
    w_t = W.T
    out = None
    for c in range(NSPLIT):
        emb2_c = _sc_gather(table, idx_perm, Q, c * qn, qn)
        out = _tc_project(emb2_c, out, w_t, b, Q, c * qn)
    return out.reshape(L, Bb, O).transpose(1, 0, 2)


# speedup vs baseline: 1.0290x; 1.0005x over previous
import functools

import jax
import jax.numpy as jnp
from jax import lax
from jax.experimental import pallas as pl
from jax.experimental.pallas import tpu as pltpu
from jax.experimental.pallas import tpu_sc as plsc

NSPLIT = 2


def _sc_gather(table, idx_flat, Q, c0, qn, chunk=400):
    V, D = table.shape
    info = plsc.get_sparse_core_info()
    nw = info.num_cores * info.num_subcores
    g_per_w = qn // nw
    n_chunks = g_per_w // chunk
    mesh = plsc.VectorSubcoreMesh(core_axis_name="c", subcore_axis_name="s")

    @functools.partial(
        pl.kernel,
        mesh=mesh,
        out_type=jax.ShapeDtypeStruct((qn, 4 * D), jnp.float32),
        scratch_types=[
            pltpu.VMEM((2, 4 * chunk), jnp.int32),
            pltpu.VMEM((2, 4 * chunk, D), jnp.float32),
            pltpu.SemaphoreType.DMA((2,)),
            pltpu.SemaphoreType.DMA((2,)),
        ],
        compiler_params=pltpu.CompilerParams(use_tc_tiling_on_sc=False),
    )
    def gather_kernel(table_hbm, idx_hbm, out_hbm, idx_v, rows_v, gsem, wsem):
        wid = lax.axis_index("s") * info.num_cores + lax.axis_index("c")
        base = wid * g_per_w

        def idx_copy(i, s):
            g0 = base + i * chunk
            for r in range(4):
                pltpu.sync_copy(
                    idx_hbm.at[pl.ds(r * Q + c0 + g0, chunk)],
                    idx_v.at[s, pl.ds(r * chunk, chunk)],
                )

        def fire_gather(s):
            for r in range(4):
                pltpu.async_copy(
                    table_hbm.at[idx_v.at[s, pl.ds(r * chunk, chunk)]],
                    rows_v.at[s, pl.ds(r * chunk, chunk)],
                    gsem.at[s],
                )

        def wait_gather(s):
            for r in range(4):
                pltpu.make_async_copy(
                    table_hbm.at[idx_v.at[s, pl.ds(r * chunk, chunk)]],
                    rows_v.at[s, pl.ds(r * chunk, chunk)],
                    gsem.at[s],
                ).wait()

        def write_copies(i, s):
            g0 = base + i * chunk
            return [
                pltpu.make_async_copy(
                    rows_v.at[s, pl.ds(r * chunk, chunk)],
                    out_hbm.at[pl.ds(g0, chunk), pl.ds(r * D, D)],
                    wsem.at[s],
                )
                for r in range(4)
            ]

        idx_copy(0, 0)
        fire_gather(0)
        if n_chunks > 1:
            idx_copy(1, 1)
            fire_gather(1)

        def body(i, carry):
            slot = lax.rem(i, 2)
            wait_gather(slot)
            for c in write_copies(i, slot):
                c.start()

            @pl.when(i + 2 < n_chunks)
            def _prep_next():
                idx_copy(i + 2, slot)
                for c in write_copies(i, slot):
                    c.wait()
                fire_gather(slot)

            @pl.when(i + 2 >= n_chunks)
            def _tail():
                for c in write_copies(i, slot):
                    c.wait()

            return carry

        lax.fori_loop(0, n_chunks, body, 0)

    return gather_kernel(table, idx_flat)


def _tc_project(emb2, prev, w_t, bias, Q, c0, block_m=4096, nbuf=4):
    qn = emb2.shape[0]
    D, O = w_t.shape
    nblk = qn // block_m

    def body(emb_ref, prev_ref, wt_ref, b_ref, out_ref, acc, sems):
        del prev_ref
        i = pl.program_id(0)
        slot = lax.rem(i, nbuf)
        row0 = c0 + i * block_m

        def _mk(s):
            return pltpu.make_async_copy(
                acc.at[s],
                out_ref.at[:, pl.ds(row0 + (s - slot) * block_m, block_m), :],
                sems.at[s],
            )

        @pl.when(i >= nbuf)
        def _wait_old():
            pltpu.make_async_copy(
                acc.at[slot],
                out_ref.at[:, pl.ds(row0 - nbuf * block_m, block_m), :],
                sems.at[slot],
            ).wait()

        for r in range(4):
            acc[slot, r] = (
                jnp.dot(
                    emb_ref[:, r * D : (r + 1) * D],
                    wt_ref[...],
                    preferred_element_type=jnp.float32,
                )
                + b_ref[...]
            )
        _mk(slot).start()

        @pl.when(i == nblk - 1)
        def _drain():
            for k in range(nbuf):
                s = lax.rem(i - k, nbuf)
                pltpu.make_async_copy(
                    acc.at[s],
                    out_ref.at[:, pl.ds(row0 - k * block_m, block_m), :],
                    sems.at[s],
                ).wait()

    args = [emb2] + ([prev] if prev is not None else []) + [w_t, bias.reshape(1, O)]
    prev_specs = [pl.BlockSpec(memory_space=pl.ANY)] if prev is not None else []
    if prev is None:
        def body0(emb_ref, wt_ref, b_ref, out_ref, acc, sems):
            body(emb_ref, None, wt_ref, b_ref, out_ref, acc, sems)
        kernel_body = body0
        aliases = {}
    else:
        kernel_body = body
        aliases = {1: 0}

    return pl.pallas_call(
        kernel_body,
        grid=(nblk,),
        in_specs=[pl.BlockSpec((block_m, 4 * D), lambda i: (i, 0))]
        + prev_specs
        + [
            pl.BlockSpec((D, O), lambda i: (0, 0)),
            pl.BlockSpec((1, O), lambda i: (0, 0)),
        ],
        out_specs=pl.BlockSpec(memory_space=pl.ANY),
        out_shape=jax.ShapeDtypeStruct((4, Q, O), jnp.float32),
        scratch_shapes=[
            pltpu.VMEM((nbuf, 4, block_m, O), jnp.float32),
            pltpu.SemaphoreType.DMA((nbuf,)),
        ],
        input_output_aliases=aliases,
    )(*args)


def kernel(idx, table, W, b):
    Bb, L = idx.shape
    O = W.shape[0]
    B = Bb * L
    Q = B // 4
    qn = Q // NSPLIT
    idx_perm = idx.T.reshape(-1).astype(jnp.int32)
    w_t = W.T
    out = None
    for c in range(NSPLIT):
        emb2_c = _sc_gather(table, idx_perm, Q, c * qn, qn)
        out = _tc_project(emb2_c, out, w_t, b, Q, c * qn)
    return out.reshape(L, Bb, O).transpose(1, 0, 2)
